# Initial kernel scaffold; baseline (speedup 1.0000x reference)
import functools

import jax
import jax.numpy as jnp
from jax import lax
from jax.experimental import pallas as pl
from jax.experimental.pallas import tpu as pltpu
from jax.experimental.pallas import tpu_sc as plsc

N_NODES = 50000
N_EDGES = 1600000
NUM_GRAPHS = 512
IN_FEATS = 3
HIDDEN = 16
NUM_CLASSES = 2
K_HOPS = 10
ALPHA = 0.1

NC = 2
NS = 16
NW = NC * NS
CH = 128
KC = 8
NP = 51200
EP = NW * 51200
EW = EP // NW
NB = EW // (KC * CH)
NTS = NP // NS
CN = 2048
NPB = NP // CN

_sc_mesh = plsc.VectorSubcoreMesh(core_axis_name="c", subcore_axis_name="s")


def _make_deg_kernel():
  @functools.partial(
      pl.kernel,
      out_type=jax.ShapeDtypeStruct((NC * NP, 1), jnp.float32),
      mesh=_sc_mesh,
      scratch_types=[
          pltpu.VMEM((KC, CH), jnp.int32),
          pltpu.VMEM((CH, 1), jnp.float32),
          pltpu.VMEM_SHARED((NP, 1), jnp.float32),
      ],
  )
  def deg_kernel(dst_hbm, ones_hbm, zeros_hbm, out_hbm, dstv, onesv, acc):
    c = lax.axis_index("c")
    s = lax.axis_index("s")
    wid = s * NC + c
    pltpu.sync_copy(zeros_hbm, acc.at[pl.ds(s * NTS, NTS)])
    pltpu.sync_copy(ones_hbm, onesv)
    plsc.subcore_barrier()

    def body(g, carry):
      row0 = wid * (EW // CH) + g * KC
      pltpu.sync_copy(dst_hbm.at[pl.ds(row0, KC)], dstv)
      for j in range(KC):
        pltpu.sync_copy(onesv, acc.at[dstv.at[j]], add=True)
      return carry

    lax.fori_loop(0, NB, body, 0)
    plsc.subcore_barrier()
    pltpu.sync_copy(acc.at[pl.ds(s * NTS, NTS)],
                    out_hbm.at[pl.ds(c * NP + s * NTS, NTS)])

  return deg_kernel


def _make_prop_kernel(width):
  @functools.partial(
      pl.kernel,
      out_type=jax.ShapeDtypeStruct((NC * NP, width), jnp.float32),
      mesh=_sc_mesh,
      scratch_types=[
          pltpu.VMEM((KC, CH), jnp.int32),
          pltpu.VMEM((KC, CH), jnp.int32),
          pltpu.VMEM((KC * CH, width), jnp.float32),
          pltpu.VMEM_SHARED((NP, width), jnp.float32),
          pltpu.SemaphoreType.DMA,
      ],
  )
  def prop_kernel(src_hbm, dst_hbm, table_hbm, zeros_hbm, out_hbm,
                  srcv, dstv, rows, acc, gsem):
    c = lax.axis_index("c")
    s = lax.axis_index("s")
    wid = s * NC + c
    pltpu.sync_copy(zeros_hbm, acc.at[pl.ds(s * NTS, NTS)])
    plsc.subcore_barrier()

    def body(g, carry):
      row0 = wid * (EW // CH) + g * KC
      pltpu.sync_copy(src_hbm.at[pl.ds(row0, KC)], srcv)
      pltpu.sync_copy(dst_hbm.at[pl.ds(row0, KC)], dstv)
      descs = []
      for j in range(KC):
        descs.append(pltpu.async_copy(table_hbm.at[srcv.at[j]],
                                      rows.at[pl.ds(j * CH, CH)], gsem))
      for d in descs:
        d.wait()
      for j in range(KC):
        pltpu.sync_copy(rows.at[pl.ds(j * CH, CH)], acc.at[dstv.at[j]],
                        add=True)
      return carry

    lax.fori_loop(0, NB, body, 0)
    plsc.subcore_barrier()
    pltpu.sync_copy(acc.at[pl.ds(s * NTS, NTS)],
                    out_hbm.at[pl.ds(c * NP + s * NTS, NTS)])

  return prop_kernel


_deg_kernel = _make_deg_kernel()
_prop3_kernel = _make_prop_kernel(IN_FEATS)
_prop2_kernel = _make_prop_kernel(NUM_CLASSES)


def _tc_norm_kernel(degp_ref, x_ref, dinv_ref, y1_ref):
  deg = degp_ref[0, :] + degp_ref[1, :] + 1.0
  dinv = lax.rsqrt(deg)
  dinv_ref[0, :] = dinv
  y1_ref[...] = x_ref[...] * dinv[:, None]


def _tc_dense_kernel(s1_ref, y1_ref, dinv_ref, w1_ref, b1_ref, wl_ref,
                     bl_ref, w2_ref, y2_ref):
  dinv = dinv_ref[...]
  agg = (s1_ref[0] + s1_ref[1] + y1_ref[...]) * dinv
  h = jnp.dot(w1_ref[...], agg, preferred_element_type=jnp.float32)
  h = jnp.maximum(h + b1_ref[...], 0.0)
  for _ in range(K_HOPS):
    hw = jnp.dot(wl_ref[...], h, preferred_element_type=jnp.float32)
    h = ALPHA * jnp.maximum(hw + bl_ref[...], 0.0) + (1.0 - ALPHA) * h
  z = jnp.dot(w2_ref[...], h, preferred_element_type=jnp.float32)
  y2_ref[...] = z * dinv


def _tc_pool_kernel(s2_ref, y2_ref, dinv_ref, b2_ref, batch_ref,
                    sums_ref, logp_ref):
  i = pl.program_id(0)
  u = (s2_ref[0] + s2_ref[1] + y2_ref[...]) * dinv_ref[...] + b2_ref[...]
  bvec = batch_ref[0, :]
  oh = (bvec[:, None] ==
        lax.broadcasted_iota(jnp.int32, (CN, NUM_GRAPHS), 1))
  oh = oh.astype(jnp.float32)
  aug = jnp.concatenate([u, jnp.ones((1, CN), jnp.float32)], axis=0)
  part = jnp.dot(aug, oh, preferred_element_type=jnp.float32)

  @pl.when(i == 0)
  def _():
    sums_ref[...] = part

  @pl.when(i > 0)
  def _():
    sums_ref[...] = sums_ref[...] + part

  @pl.when(i == NPB - 1)
  def _():
    sums = sums_ref[...]
    cnt = jnp.maximum(sums[2:3, :], 1.0)
    p = sums[0:2, :] / cnt
    m = jnp.max(p, axis=0, keepdims=True)
    lse = m + jnp.log(jnp.sum(jnp.exp(p - m), axis=0, keepdims=True))
    logp_ref[...] = p - lse


def kernel(x, edge_index, batch, W1, b1, Wlin, blin, W2, b2):
  f32 = jnp.float32
  src = edge_index[0].astype(jnp.int32)
  dst = edge_index[1].astype(jnp.int32)
  pad_ids = (jnp.arange(EP - N_EDGES, dtype=jnp.int32) % (NP - N_NODES)
             ) + N_NODES
  src2d = jnp.concatenate([src, pad_ids]).reshape(EP // CH, CH)
  dst2d = jnp.concatenate([dst, pad_ids]).reshape(EP // CH, CH)

  ones_col = jnp.ones((CH, 1), f32)
  zeros1 = jnp.zeros((NTS, 1), f32)
  zeros2 = jnp.zeros((NTS, NUM_CLASSES), f32)
  zeros3 = jnp.zeros((NTS, IN_FEATS), f32)

  degp = _deg_kernel(dst2d, ones_col, zeros1)

  xp = jnp.concatenate([x, jnp.zeros((NP - N_NODES, IN_FEATS), f32)])
  dinv, y1 = pl.pallas_call(
      _tc_norm_kernel,
      out_shape=(jax.ShapeDtypeStruct((1, NP), f32),
                 jax.ShapeDtypeStruct((NP, IN_FEATS), f32)),
  )(degp.reshape(2, NP), xp)

  s1 = _prop3_kernel(src2d, dst2d, y1, zeros3)

  y2t = pl.pallas_call(
      _tc_dense_kernel,
      out_shape=jax.ShapeDtypeStruct((NUM_CLASSES, NP), f32),
  )(s1.reshape(2, NP, IN_FEATS).transpose(0, 2, 1), y1.T, dinv,
    W1.T, b1.reshape(HIDDEN, 1), Wlin.T, blin.reshape(HIDDEN, 1), W2.T)

  y2 = y2t.T
  s2 = _prop2_kernel(src2d, dst2d, y2, zeros2)

  batch_p = jnp.concatenate([
      batch.astype(jnp.int32),
      jnp.full((NP - N_NODES,), NUM_GRAPHS, jnp.int32)]).reshape(1, NP)
  s2t = s2.reshape(2, NP, NUM_CLASSES).transpose(0, 2, 1)
  _, logp = pl.pallas_call(
      _tc_pool_kernel,
      grid=(NPB,),
      in_specs=[
          pl.BlockSpec((2, NUM_CLASSES, CN), lambda i: (0, 0, i)),
          pl.BlockSpec((NUM_CLASSES, CN), lambda i: (0, i)),
          pl.BlockSpec((1, CN), lambda i: (0, i)),
          pl.BlockSpec((NUM_CLASSES, 1), lambda i: (0, 0)),
          pl.BlockSpec((1, CN), lambda i: (0, i)),
      ],
      out_specs=(
          pl.BlockSpec((3, NUM_GRAPHS), lambda i: (0, 0)),
          pl.BlockSpec((NUM_CLASSES, NUM_GRAPHS), lambda i: (0, 0)),
      ),
      out_shape=(jax.ShapeDtypeStruct((3, NUM_GRAPHS), f32),
                 jax.ShapeDtypeStruct((NUM_CLASSES, NUM_GRAPHS), f32)),
  )(s2t, y2t, dinv, b2.reshape(NUM_CLASSES, 1), batch_p)
  return logp.T


# speedup vs baseline: 52.4581x; 52.4581x over previous
import functools

import jax
import jax.numpy as jnp
from jax import lax
from jax.experimental import pallas as pl
from jax.experimental.pallas import tpu as pltpu
from jax.experimental.pallas import tpu_sc as plsc

N_NODES = 50000
N_EDGES = 1600000
NUM_GRAPHS = 512
IN_FEATS = 3
HIDDEN = 16
NUM_CLASSES = 2
K_HOPS = 10
ALPHA = 0.1

NC = 2
NS = 16
NW = NC * NS
CH = 128
KC = 8
NP = 51200
EP = NW * 51200
EW = EP // NW
NB = EW // (KC * CH)
NTS = NP // NS
CN = 2048
NPB = NP // CN

_sc_mesh = plsc.VectorSubcoreMesh(core_axis_name="c", subcore_axis_name="s")


def _make_deg_kernel():
  @functools.partial(
      pl.kernel,
      out_type=jax.ShapeDtypeStruct((NC * NP,), jnp.float32),
      mesh=_sc_mesh,
      scratch_types=[
          pltpu.VMEM((KC, 1, CH), jnp.int32),
          pltpu.VMEM((CH,), jnp.float32),
          pltpu.VMEM_SHARED((NP,), jnp.float32),
      ],
  )
  def deg_kernel(dst_hbm, ones_hbm, zeros_hbm, out_hbm, dstv, onesv, acc):
    c = lax.axis_index("c")
    s = lax.axis_index("s")
    wid = s * NC + c
    pltpu.sync_copy(zeros_hbm, acc.at[pl.ds(s * NTS, NTS)])
    pltpu.sync_copy(ones_hbm, onesv)
    plsc.subcore_barrier()

    def body(g, carry):
      row0 = wid * (EW // CH) + g * KC
      pltpu.sync_copy(dst_hbm.at[pl.ds(row0, KC)], dstv)
      for j in range(KC):
        pltpu.sync_copy(onesv, acc.at[dstv.at[j, 0]], add=True)
      return carry

    lax.fori_loop(0, NB, body, 0)
    plsc.subcore_barrier()
    pltpu.sync_copy(acc.at[pl.ds(s * NTS, NTS)],
                    out_hbm.at[pl.ds(c * NP + s * NTS, NTS)])

  return deg_kernel


def _make_prop_kernel(width):
  out_types = tuple(jax.ShapeDtypeStruct((NC * NP,), jnp.float32)
                    for _ in range(width))
  scratch = [
      pltpu.VMEM((KC, 1, CH), jnp.int32),
      pltpu.VMEM((KC, 1, CH), jnp.int32),
  ]
  scratch += [pltpu.VMEM((KC * CH,), jnp.float32) for _ in range(width)]
  scratch += [pltpu.VMEM_SHARED((NP,), jnp.float32) for _ in range(width)]
  scratch.append(pltpu.SemaphoreType.DMA)

  @functools.partial(
      pl.kernel,
      out_type=out_types,
      mesh=_sc_mesh,
      scratch_types=scratch,
  )
  def prop_kernel(src_hbm, dst_hbm, *rest):
    tabs = rest[:width]
    zeros_hbm = rest[width]
    outs = rest[width + 1:width + 1 + width]
    srcv, dstv = rest[width + 1 + width:width + 3 + width]
    cols = rest[width + 3 + width:width + 3 + 2 * width]
    accs = rest[width + 3 + 2 * width:width + 3 + 3 * width]
    gsem = rest[-1]

    c = lax.axis_index("c")
    s = lax.axis_index("s")
    wid = s * NC + c
    for acc in accs:
      pltpu.sync_copy(zeros_hbm, acc.at[pl.ds(s * NTS, NTS)])
    plsc.subcore_barrier()

    def body(g, carry):
      row0 = wid * (EW // CH) + g * KC
      pltpu.sync_copy(src_hbm.at[pl.ds(row0, KC)], srcv)
      pltpu.sync_copy(dst_hbm.at[pl.ds(row0, KC)], dstv)
      descs = []
      for j in range(KC):
        for w in range(width):
          descs.append(pltpu.async_copy(
              tabs[w].at[srcv.at[j, 0]],
              cols[w].at[pl.ds(j * CH, CH)], gsem))
      for d in descs:
        d.wait()
      for j in range(KC):
        for w in range(width):
          pltpu.sync_copy(cols[w].at[pl.ds(j * CH, CH)],
                          accs[w].at[dstv.at[j, 0]], add=True)
      return carry

    lax.fori_loop(0, NB, body, 0)
    plsc.subcore_barrier()
    for w in range(width):
      pltpu.sync_copy(accs[w].at[pl.ds(s * NTS, NTS)],
                      outs[w].at[pl.ds(c * NP + s * NTS, NTS)])

  return prop_kernel


_deg_kernel = _make_deg_kernel()
_prop3_kernel = _make_prop_kernel(IN_FEATS)
_prop2_kernel = _make_prop_kernel(NUM_CLASSES)


def _tc_norm_kernel(degp_ref, xt_ref, dinv_ref, y1t_ref):
  deg = degp_ref[0, :] + degp_ref[1, :] + 1.0
  dinv = lax.rsqrt(deg)
  dinv_ref[0, :] = dinv
  y1t_ref[...] = xt_ref[...] * dinv[None, :]


def _tc_dense_kernel(s1_ref, y1t_ref, dinv_ref, w1_ref, b1_ref, wl_ref,
                     bl_ref, w2_ref, y2t_ref):
  dinv = dinv_ref[...]
  s1 = s1_ref[...]
  agg = (s1[:, 0, :] + s1[:, 1, :] + y1t_ref[...]) * dinv
  h = jnp.dot(w1_ref[...], agg, preferred_element_type=jnp.float32)
  h = jnp.maximum(h + b1_ref[...], 0.0)
  for _ in range(K_HOPS):
    hw = jnp.dot(wl_ref[...], h, preferred_element_type=jnp.float32)
    h = ALPHA * jnp.maximum(hw + bl_ref[...], 0.0) + (1.0 - ALPHA) * h
  z = jnp.dot(w2_ref[...], h, preferred_element_type=jnp.float32)
  y2t_ref[...] = z * dinv


def _tc_pool_kernel(s2_ref, y2t_ref, dinv_ref, b2_ref, batch_ref,
                    sums_ref, logp_ref):
  i = pl.program_id(0)
  s2 = s2_ref[...]
  u = (s2[:, 0, :] + s2[:, 1, :] + y2t_ref[...]) * dinv_ref[...] + b2_ref[...]
  bvec = batch_ref[0, :]
  oh = (bvec[:, None] ==
        lax.broadcasted_iota(jnp.int32, (CN, NUM_GRAPHS), 1))
  oh = oh.astype(jnp.float32)
  aug = jnp.concatenate([u, jnp.ones((1, CN), jnp.float32)], axis=0)
  part = jnp.dot(aug, oh, preferred_element_type=jnp.float32)

  @pl.when(i == 0)
  def _():
    sums_ref[...] = part

  @pl.when(i > 0)
  def _():
    sums_ref[...] = sums_ref[...] + part

  @pl.when(i == NPB - 1)
  def _():
    sums = sums_ref[...]
    cnt = jnp.maximum(sums[2:3, :], 1.0)
    p = sums[0:2, :] / cnt
    m = jnp.max(p, axis=0, keepdims=True)
    lse = m + jnp.log(jnp.sum(jnp.exp(p - m), axis=0, keepdims=True))
    logp_ref[...] = p - lse


def kernel(x, edge_index, batch, W1, b1, Wlin, blin, W2, b2):
  f32 = jnp.float32
  src = edge_index[0].astype(jnp.int32)
  dst = edge_index[1].astype(jnp.int32)
  pad_ids = (jnp.arange(EP - N_EDGES, dtype=jnp.int32) % (NP - N_NODES)
             ) + N_NODES
  src3d = jnp.concatenate([src, pad_ids]).reshape(EP // CH, 1, CH)
  dst3d = jnp.concatenate([dst, pad_ids]).reshape(EP // CH, 1, CH)

  ones_v = jnp.ones((CH,), f32)
  zeros_t = jnp.zeros((NTS,), f32)

  degp = _deg_kernel(dst3d, ones_v, zeros_t)

  xt = jnp.concatenate([x, jnp.zeros((NP - N_NODES, IN_FEATS), f32)]).T
  dinv, y1t = pl.pallas_call(
      _tc_norm_kernel,
      out_shape=(jax.ShapeDtypeStruct((1, NP), f32),
                 jax.ShapeDtypeStruct((IN_FEATS, NP), f32)),
  )(degp.reshape(2, NP), xt)

  s1c = _prop3_kernel(src3d, dst3d, y1t[0], y1t[1], y1t[2], zeros_t)
  s1 = jnp.stack([p.reshape(2, NP) for p in s1c])

  y2t = pl.pallas_call(
      _tc_dense_kernel,
      out_shape=jax.ShapeDtypeStruct((NUM_CLASSES, NP), f32),
  )(s1, y1t, dinv, W1.T, b1.reshape(HIDDEN, 1), Wlin.T,
    blin.reshape(HIDDEN, 1), W2.T)

  s2c = _prop2_kernel(src3d, dst3d, y2t[0], y2t[1], zeros_t)
  s2 = jnp.stack([p.reshape(2, NP) for p in s2c])

  batch_p = jnp.concatenate([
      batch.astype(jnp.int32),
      jnp.full((NP - N_NODES,), NUM_GRAPHS, jnp.int32)]).reshape(1, NP)
  _, logp = pl.pallas_call(
      _tc_pool_kernel,
      grid=(NPB,),
      in_specs=[
          pl.BlockSpec((2, 2, CN), lambda i: (0, 0, i)),
          pl.BlockSpec((NUM_CLASSES, CN), lambda i: (0, i)),
          pl.BlockSpec((1, CN), lambda i: (0, i)),
          pl.BlockSpec((NUM_CLASSES, 1), lambda i: (0, 0)),
          pl.BlockSpec((1, CN), lambda i: (0, i)),
      ],
      out_specs=(
          pl.BlockSpec((3, NUM_GRAPHS), lambda i: (0, 0)),
          pl.BlockSpec((NUM_CLASSES, NUM_GRAPHS), lambda i: (0, 0)),
      ),
      out_shape=(jax.ShapeDtypeStruct((3, NUM_GRAPHS), f32),
                 jax.ShapeDtypeStruct((NUM_CLASSES, NUM_GRAPHS), f32)),
  )(s2, y2t, dinv, b2.reshape(NUM_CLASSES, 1), batch_p)
  return logp.T


# speedup vs baseline: 78.3496x; 1.4936x over previous
import functools

import jax
import jax.numpy as jnp
from jax import lax
from jax.experimental import pallas as pl
from jax.experimental.pallas import tpu as pltpu
from jax.experimental.pallas import tpu_sc as plsc

N_NODES = 50000
N_EDGES = 1600000
NUM_GRAPHS = 512
IN_FEATS = 3
HIDDEN = 16
NUM_CLASSES = 2
K_HOPS = 10
ALPHA = 0.1

NC = 2
NS = 16
NW = NC * NS
CH = 128
KC = 8
NP = 51200
EP = NW * 51200
EW = EP // NW
NB = EW // (KC * CH)
NTS = NP // NS
CN = 2048
NPB = NP // CN

_sc_mesh = plsc.VectorSubcoreMesh(core_axis_name="c", subcore_axis_name="s")


NBUF = 2
NB2 = NB // NBUF


def _make_deg_kernel():
  scratch = [pltpu.VMEM((KC, 1, CH), jnp.int32) for _ in range(NBUF)]
  scratch.append(pltpu.VMEM((CH,), jnp.float32))
  scratch.append(pltpu.VMEM_SHARED((NP,), jnp.float32))
  scratch += [pltpu.SemaphoreType.DMA for _ in range(NBUF)]

  @functools.partial(
      pl.kernel,
      out_type=jax.ShapeDtypeStruct((NC * NP,), jnp.float32),
      mesh=_sc_mesh,
      scratch_types=scratch,
  )
  def deg_kernel(dst_hbm, ones_hbm, zeros_hbm, out_hbm, *rest):
    dstv = rest[:NBUF]
    onesv = rest[NBUF]
    acc = rest[NBUF + 1]
    ssem = rest[NBUF + 2:]
    c = lax.axis_index("c")
    s = lax.axis_index("s")
    wid = s * NC + c
    pltpu.sync_copy(zeros_hbm, acc.at[pl.ds(s * NTS, NTS)])
    pltpu.sync_copy(ones_hbm, onesv)
    plsc.subcore_barrier()

    def body(g, carry):
      descs = []
      for b in range(NBUF):
        row0 = wid * (EW // CH) + (g * NBUF + b) * KC
        pltpu.sync_copy(dst_hbm.at[pl.ds(row0, KC)], dstv[b])
        for j in range(KC):
          descs.append(pltpu.async_copy(
              onesv, acc.at[dstv[b].at[j, 0]], ssem[b], add=True))
      for d in descs:
        d.wait()
      return carry

    lax.fori_loop(0, NB2, body, 0)
    plsc.subcore_barrier()
    pltpu.sync_copy(acc.at[pl.ds(s * NTS, NTS)],
                    out_hbm.at[pl.ds(c * NP + s * NTS, NTS)])

  return deg_kernel


def _make_prop_kernel(width):
  out_types = tuple(jax.ShapeDtypeStruct((NC * NP,), jnp.float32)
                    for _ in range(width))
  scratch = [pltpu.VMEM((KC, 1, CH), jnp.int32)
             for _ in range(2 * NBUF)]
  scratch += [pltpu.VMEM((KC * CH,), jnp.float32)
              for _ in range(NBUF * width)]
  scratch += [pltpu.VMEM_SHARED((NP,), jnp.float32) for _ in range(width)]
  scratch += [pltpu.SemaphoreType.DMA for _ in range(2 * NBUF)]

  @functools.partial(
      pl.kernel,
      out_type=out_types,
      mesh=_sc_mesh,
      scratch_types=scratch,
  )
  def prop_kernel(src_hbm, dst_hbm, *rest):
    tabs = rest[:width]
    zeros_hbm = rest[width]
    k = width + 1
    outs = rest[k:k + width]; k += width
    srcv = rest[k:k + NBUF]; k += NBUF
    dstv = rest[k:k + NBUF]; k += NBUF
    cols = [rest[k + b * width:k + (b + 1) * width] for b in range(NBUF)]
    k += NBUF * width
    accs = rest[k:k + width]; k += width
    gsem = rest[k:k + NBUF]; k += NBUF
    ssem = rest[k:k + NBUF]

    c = lax.axis_index("c")
    s = lax.axis_index("s")
    wid = s * NC + c
    for acc in accs:
      pltpu.sync_copy(zeros_hbm, acc.at[pl.ds(s * NTS, NTS)])
    plsc.subcore_barrier()

    def body(g, carry):
      gdescs = [[] for _ in range(NBUF)]
      for b in range(NBUF):
        row0 = wid * (EW // CH) + (g * NBUF + b) * KC
        pltpu.sync_copy(src_hbm.at[pl.ds(row0, KC)], srcv[b])
        pltpu.sync_copy(dst_hbm.at[pl.ds(row0, KC)], dstv[b])
        for j in range(KC):
          for w in range(width):
            gdescs[b].append(pltpu.async_copy(
                tabs[w].at[srcv[b].at[j, 0]],
                cols[b][w].at[pl.ds(j * CH, CH)], gsem[b]))
      sdescs = []
      for b in range(NBUF):
        for d in gdescs[b]:
          d.wait()
        for j in range(KC):
          for w in range(width):
            sdescs.append(pltpu.async_copy(
                cols[b][w].at[pl.ds(j * CH, CH)],
                accs[w].at[dstv[b].at[j, 0]], ssem[b], add=True))
      for d in sdescs:
        d.wait()
      return carry

    lax.fori_loop(0, NB2, body, 0)
    plsc.subcore_barrier()
    for w in range(width):
      pltpu.sync_copy(accs[w].at[pl.ds(s * NTS, NTS)],
                      outs[w].at[pl.ds(c * NP + s * NTS, NTS)])

  return prop_kernel


_deg_kernel = _make_deg_kernel()
_prop3_kernel = _make_prop_kernel(IN_FEATS)
_prop2_kernel = _make_prop_kernel(NUM_CLASSES)


def _tc_norm_kernel(degp_ref, xt_ref, dinv_ref, y1t_ref):
  deg = degp_ref[0, :] + degp_ref[1, :] + 1.0
  dinv = lax.rsqrt(deg)
  dinv_ref[0, :] = dinv
  y1t_ref[...] = xt_ref[...] * dinv[None, :]


def _tc_dense_kernel(s1_ref, y1t_ref, dinv_ref, w1_ref, b1_ref, wl_ref,
                     bl_ref, w2_ref, y2t_ref):
  dinv = dinv_ref[...]
  s1 = s1_ref[...]
  agg = (s1[:, 0, :] + s1[:, 1, :] + y1t_ref[...]) * dinv
  h = jnp.dot(w1_ref[...], agg, preferred_element_type=jnp.float32)
  h = jnp.maximum(h + b1_ref[...], 0.0)
  for _ in range(K_HOPS):
    hw = jnp.dot(wl_ref[...], h, preferred_element_type=jnp.float32)
    h = ALPHA * jnp.maximum(hw + bl_ref[...], 0.0) + (1.0 - ALPHA) * h
  z = jnp.dot(w2_ref[...], h, preferred_element_type=jnp.float32)
  y2t_ref[...] = z * dinv


def _tc_pool_kernel(s2_ref, y2t_ref, dinv_ref, b2_ref, batch_ref,
                    sums_ref, logp_ref):
  i = pl.program_id(0)
  s2 = s2_ref[...]
  u = (s2[:, 0, :] + s2[:, 1, :] + y2t_ref[...]) * dinv_ref[...] + b2_ref[...]
  bvec = batch_ref[0, :]
  oh = (bvec[:, None] ==
        lax.broadcasted_iota(jnp.int32, (CN, NUM_GRAPHS), 1))
  oh = oh.astype(jnp.float32)
  aug = jnp.concatenate([u, jnp.ones((1, CN), jnp.float32)], axis=0)
  part = jnp.dot(aug, oh, preferred_element_type=jnp.float32)

  @pl.when(i == 0)
  def _():
    sums_ref[...] = part

  @pl.when(i > 0)
  def _():
    sums_ref[...] = sums_ref[...] + part

  @pl.when(i == NPB - 1)
  def _():
    sums = sums_ref[...]
    cnt = jnp.maximum(sums[2:3, :], 1.0)
    p = sums[0:2, :] / cnt
    m = jnp.max(p, axis=0, keepdims=True)
    lse = m + jnp.log(jnp.sum(jnp.exp(p - m), axis=0, keepdims=True))
    logp_ref[...] = p - lse


def kernel(x, edge_index, batch, W1, b1, Wlin, blin, W2, b2):
  f32 = jnp.float32
  src = edge_index[0].astype(jnp.int32)
  dst = edge_index[1].astype(jnp.int32)
  pad_ids = (jnp.arange(EP - N_EDGES, dtype=jnp.int32) % (NP - N_NODES)
             ) + N_NODES
  src3d = jnp.concatenate([src, pad_ids]).reshape(EP // CH, 1, CH)
  dst3d = jnp.concatenate([dst, pad_ids]).reshape(EP // CH, 1, CH)

  ones_v = jnp.ones((CH,), f32)
  zeros_t = jnp.zeros((NTS,), f32)

  degp = _deg_kernel(dst3d, ones_v, zeros_t)

  xt = jnp.concatenate([x, jnp.zeros((NP - N_NODES, IN_FEATS), f32)]).T
  dinv, y1t = pl.pallas_call(
      _tc_norm_kernel,
      out_shape=(jax.ShapeDtypeStruct((1, NP), f32),
                 jax.ShapeDtypeStruct((IN_FEATS, NP), f32)),
  )(degp.reshape(2, NP), xt)

  s1c = _prop3_kernel(src3d, dst3d, y1t[0], y1t[1], y1t[2], zeros_t)
  s1 = jnp.stack([p.reshape(2, NP) for p in s1c])

  y2t = pl.pallas_call(
      _tc_dense_kernel,
      out_shape=jax.ShapeDtypeStruct((NUM_CLASSES, NP), f32),
  )(s1, y1t, dinv, W1.T, b1.reshape(HIDDEN, 1), Wlin.T,
    blin.reshape(HIDDEN, 1), W2.T)

  s2c = _prop2_kernel(src3d, dst3d, y2t[0], y2t[1], zeros_t)
  s2 = jnp.stack([p.reshape(2, NP) for p in s2c])

  batch_p = jnp.concatenate([
      batch.astype(jnp.int32),
      jnp.full((NP - N_NODES,), NUM_GRAPHS, jnp.int32)]).reshape(1, NP)
  _, logp = pl.pallas_call(
      _tc_pool_kernel,
      grid=(NPB,),
      in_specs=[
          pl.BlockSpec((2, 2, CN), lambda i: (0, 0, i)),
          pl.BlockSpec((NUM_CLASSES, CN), lambda i: (0, i)),
          pl.BlockSpec((1, CN), lambda i: (0, i)),
          pl.BlockSpec((NUM_CLASSES, 1), lambda i: (0, 0)),
          pl.BlockSpec((1, CN), lambda i: (0, i)),
      ],
      out_specs=(
          pl.BlockSpec((3, NUM_GRAPHS), lambda i: (0, 0)),
          pl.BlockSpec((NUM_CLASSES, NUM_GRAPHS), lambda i: (0, 0)),
      ),
      out_shape=(jax.ShapeDtypeStruct((3, NUM_GRAPHS), f32),
                 jax.ShapeDtypeStruct((NUM_CLASSES, NUM_GRAPHS), f32)),
  )(s2, y2t, dinv, b2.reshape(NUM_CLASSES, 1), batch_p)
  return logp.T
